# Initial kernel scaffold; baseline (speedup 1.0000x reference)
import jax
import jax.numpy as jnp
from jax.experimental import pallas as pl


def kernel(token_ids, weight):
    raise NotImplementedError("write your pallas kernel here")


# speedup vs baseline: 1.6870x; 1.6870x over previous
import functools

import jax
import jax.numpy as jnp
from jax import lax
from jax.experimental import pallas as pl
from jax.experimental.pallas import tpu as pltpu
from jax.experimental.pallas import tpu_sc as plsc

NC, NS = 2, 16
NW = NC * NS
CHUNK = 128
EMB = 64


@functools.partial(jax.jit, static_argnums=(0,))
def _lookup(n_chunks, idx2d, table):
    rows_per_w = n_chunks // NW
    mesh = plsc.VectorSubcoreMesh(core_axis_name="c", subcore_axis_name="s")

    @functools.partial(
        pl.kernel,
        out_type=jax.ShapeDtypeStruct((n_chunks * CHUNK, EMB), jnp.float32),
        mesh=mesh,
        scratch_types=[
            pltpu.VMEM((rows_per_w, CHUNK), jnp.int32),
            pltpu.VMEM((CHUNK, EMB), jnp.float32),
            pltpu.SemaphoreType.DMA,
        ],
        compiler_params=pltpu.CompilerParams(use_tc_tiling_on_sc=False),
    )
    def k(idx_hbm, table_hbm, out_hbm, idx_v, rows_v, sem):
        wid = lax.axis_index("s") * NC + lax.axis_index("c")
        row0 = wid * rows_per_w
        pltpu.sync_copy(idx_hbm.at[pl.ds(row0, rows_per_w)], idx_v)

        @pl.loop(0, rows_per_w)
        def _(j):
            pltpu.async_copy(table_hbm.at[idx_v.at[j]], rows_v, sem).wait()
            pltpu.sync_copy(rows_v, out_hbm.at[pl.ds((row0 + j) * CHUNK, CHUNK)])

    return k(idx2d, table)


def kernel(token_ids, weight):
    ids = token_ids.reshape(-1).astype(jnp.int32)
    n = ids.shape[0]
    idx2d = ids.reshape(n // CHUNK, CHUNK)
    out = _lookup(n // CHUNK, idx2d, weight)
    return out.reshape(*token_ids.shape, EMB)


# speedup vs baseline: 1.8738x; 1.1107x over previous
import functools

import jax
import jax.numpy as jnp
from jax import lax
from jax.experimental import pallas as pl
from jax.experimental.pallas import tpu as pltpu
from jax.experimental.pallas import tpu_sc as plsc

NC, NS = 2, 16
NW = NC * NS
CHUNK = 128
NBUF = 4
EMB = 64


@functools.partial(jax.jit, static_argnums=(0,))
def _lookup(n_chunks, idx2d, table):
    rows_per_w = n_chunks // NW
    ngroups = rows_per_w // NBUF
    grp_rows = NBUF * CHUNK
    mesh = plsc.VectorSubcoreMesh(core_axis_name="c", subcore_axis_name="s")

    @functools.partial(
        pl.kernel,
        out_type=jax.ShapeDtypeStruct((n_chunks * CHUNK, EMB), jnp.float32),
        mesh=mesh,
        scratch_types=[
            pltpu.VMEM((rows_per_w, CHUNK), jnp.int32),
            pltpu.VMEM((grp_rows, EMB), jnp.float32),
            pltpu.VMEM((grp_rows, EMB), jnp.float32),
            pltpu.SemaphoreType.DMA,
            pltpu.SemaphoreType.DMA,
            pltpu.SemaphoreType.DMA,
            pltpu.SemaphoreType.DMA,
        ],
        compiler_params=pltpu.CompilerParams(use_tc_tiling_on_sc=False),
    )
    def k(idx_hbm, table_hbm, out_hbm, idx_v, buf0, buf1, g0, g1, o0, o1):
        wid = lax.axis_index("s") * NC + lax.axis_index("c")
        row0 = wid * rows_per_w
        pltpu.sync_copy(idx_hbm.at[pl.ds(row0, rows_per_w)], idx_v)

        def gfire(grp, buf, sem):
            for b in range(NBUF):
                pltpu.async_copy(
                    table_hbm.at[idx_v.at[grp * NBUF + b]],
                    buf.at[pl.ds(b * CHUNK, CHUNK)],
                    sem,
                )

        def gdrain(buf, sem):
            for b in range(NBUF):
                pltpu.make_async_copy(
                    table_hbm.at[idx_v.at[0]],
                    buf.at[pl.ds(b * CHUNK, CHUNK)],
                    sem,
                ).wait()

        def ofire(grp, buf, sem):
            pltpu.async_copy(
                buf, out_hbm.at[pl.ds((row0 + grp * NBUF) * CHUNK, grp_rows)], sem
            )

        def odrain(buf, sem):
            pltpu.make_async_copy(
                buf, out_hbm.at[pl.ds(0, grp_rows)], sem
            ).wait()

        gfire(0, buf0, g0)

        @pl.loop(0, ngroups, step=2)
        def _(g):
            @pl.when(g > 0)
            def _():
                odrain(buf1, o1)
            gfire(g + 1, buf1, g1)
            gdrain(buf0, g0)
            ofire(g, buf0, o0)

            @pl.when(g + 2 < ngroups)
            def _():
                odrain(buf0, o0)
                gfire(g + 2, buf0, g0)
            gdrain(buf1, g1)
            ofire(g + 1, buf1, o1)

        odrain(buf0, o0)
        odrain(buf1, o1)

    return k(idx2d, table)


def kernel(token_ids, weight):
    ids = token_ids.reshape(-1).astype(jnp.int32)
    n = ids.shape[0]
    idx2d = ids.reshape(n // CHUNK, CHUNK)
    out = _lookup(n // CHUNK, idx2d, weight)
    return out.reshape(*token_ids.shape, EMB)
